# baseline (speedup 1.0000x reference)
import functools

import jax
import jax.numpy as jnp
from jax import lax
from jax.experimental import pallas as pl
from jax.experimental.pallas import tpu as pltpu
from jax.experimental.pallas import tpu_sc as plsc

N = 10000
H = 16
K = 16
L = 4
RB = 256
PADN = 10240
BIG = 1.0e30


NLANE = 128
NG = PADN // NLANE
T = 5


def _knn_body(nrows, hr_ref, hT_ref, sq_ref, idx_ref, d_ref):
    i = pl.program_id(0)
    hr = hr_ref[...]
    p = jnp.dot(hr, hT_ref[...], precision=jax.lax.Precision.HIGHEST,
                preferred_element_type=jnp.float32)
    sq_r = jnp.sum(hr * hr, axis=1)
    d = (sq_r[:, None] + sq_ref[...]) - 2.0 * p
    col = jax.lax.broadcasted_iota(jnp.int32, (RB, PADN), 1)
    row = jax.lax.broadcasted_iota(jnp.int32, (RB, PADN), 0) + i * RB
    d = jnp.where(col == row, BIG, d)
    d_ref[...] = d

    d3 = d.reshape(RB, NG, NLANE)
    g_iota = jax.lax.broadcasted_iota(jnp.int32, (RB, NG, NLANE), 1)
    lane_iota = jax.lax.broadcasted_iota(jnp.int32, (RB, NLANE), 1)
    svals, scols = [], []
    m_prev = None
    for t in range(T):
        dm = d3 if t == 0 else jnp.where(d3 > m_prev[:, None, :], d3, BIG)
        m_t = jnp.min(dm, axis=1)
        g_t = jnp.min(jnp.where(dm == m_t[:, None, :], g_iota, NG), axis=1)
        svals.append(m_t)
        scols.append(g_t * NLANE + lane_iota)
        m_prev = m_t
    W2 = T * NLANE
    S = jnp.stack(svals, axis=1).reshape(RB, W2)
    C = jnp.stack(scols, axis=1).reshape(RB, W2)

    slot = jax.lax.broadcasted_iota(jnp.int32, (RB, W2), 1)
    idxs = []
    maxlev = jnp.zeros((RB,), jnp.int32)
    for _ in range(K):
        m = jnp.min(S, axis=1)
        first = jnp.min(jnp.where(S == m[:, None], slot, W2), axis=1)
        cp = jnp.min(jnp.where(slot == first[:, None], C, PADN), axis=1)
        idxs.append(cp)
        maxlev = jnp.maximum(maxlev, first // NLANE)
        S = jnp.where(slot == first[:, None], BIG, S)
    danger = jnp.max(maxlev) >= T - 1

    @pl.when(jnp.logical_not(danger))
    def _fast():
        idx_ref[...] = jnp.stack(idxs, axis=1).astype(jnp.int32)

    @pl.when(danger)
    def _slow():
        res = []
        for _ in range(K):
            dd = d_ref[...]
            mm = jnp.min(dd, axis=1)
            ff = jnp.min(jnp.where(dd == mm[:, None], col, PADN), axis=1)
            res.append(ff)
            d_ref[...] = jnp.where(col == ff[:, None], BIG, dd)
        idx_ref[...] = jnp.stack(res, axis=1).astype(jnp.int32)


def _knn_idx_pallas(h_pad, sq_pad):
    grid = PADN // RB
    body = functools.partial(_knn_body, N)
    return pl.pallas_call(
        body,
        grid=(grid,),
        in_specs=[
            pl.BlockSpec((RB, H), lambda i: (i, 0)),
            pl.BlockSpec((H, PADN), lambda i: (0, 0)),
            pl.BlockSpec((1, PADN), lambda i: (0, 0)),
        ],
        out_specs=pl.BlockSpec((RB, K), lambda i: (i, 0)),
        out_shape=jax.ShapeDtypeStruct((PADN, K), jnp.int32),
        scratch_shapes=[pltpu.VMEM((RB, PADN), jnp.float32)],
    )(h_pad, h_pad.T, sq_pad)


NW = 32
ROWS_W = PADN // NW
CH = 128
NCH = ROWS_W * K // CH


def _sc_edge(idx3, a, g):
    mesh = plsc.VectorSubcoreMesh(core_axis_name="c", subcore_axis_name="s")

    @functools.partial(
        pl.kernel, mesh=mesh,
        out_type=jax.ShapeDtypeStruct((PADN, H), jnp.float32),
        compiler_params=pltpu.CompilerParams(use_tc_tiling_on_sc=False),
        scratch_types=[
            pltpu.VMEM((NCH, CH), jnp.int32),
            pltpu.VMEM((ROWS_W * K, H), jnp.float32),
            pltpu.VMEM((ROWS_W, H), jnp.float32),
            pltpu.VMEM((ROWS_W, H), jnp.float32),
            pltpu.SemaphoreType.DMA,
        ],
    )
    def k(idx_hbm, a_hbm, g_hbm, out_hbm, idx_v, rows_v, a_v, s_v, sem):
        wid = lax.axis_index("s") * 2 + lax.axis_index("c")
        base = wid * ROWS_W
        pltpu.sync_copy(idx_hbm.at[wid], idx_v)
        pltpu.sync_copy(a_hbm.at[pl.ds(base, ROWS_W)], a_v)

        def gat(c, carry):
            pltpu.async_copy(g_hbm.at[idx_v.at[c]],
                             rows_v.at[pl.ds(c * CH, CH)], sem).wait()
            return carry

        lax.fori_loop(0, NCH, gat, 0)

        def row(r, carry):
            a_r = a_v[r, :]
            s = jnp.maximum(a_r + rows_v[r * K, :], 0.0)
            for j in range(1, K):
                s = s + jnp.maximum(a_r + rows_v[r * K + j, :], 0.0)
            s_v[r, :] = s
            return carry

        lax.fori_loop(0, ROWS_W, row, 0)
        pltpu.sync_copy(s_v, out_hbm.at[pl.ds(base, ROWS_W)])

    return k(idx3, a, g)


def _hupd_body(s_ref, w2_ref, b2_ref, o_ref):
    o_ref[...] = jnp.dot(s_ref[...] * (1.0 / K), w2_ref[...],
                         preferred_element_type=jnp.float32) + b2_ref[...]


def _h_update_pallas(s, w2, b2):
    return pl.pallas_call(
        _hupd_body,
        out_shape=jax.ShapeDtypeStruct((PADN, H), jnp.float32),
    )(s, w2, b2[None, :])


def _final_body(hc_ref, w_ref, b_ref, o_ref):
    o_ref[...] = jnp.dot(hc_ref[...], w_ref[...],
                         preferred_element_type=jnp.float32) + b_ref[...]


def kernel(x, batch, emb_w, emb_b,
           ec0_w1, ec0_b1, ec0_w2, ec0_b2,
           ec1_w1, ec1_b1, ec1_w2, ec1_b2,
           ec2_w1, ec2_b1, ec2_w2, ec2_b2,
           ec3_w1, ec3_b1, ec3_w2, ec3_b2,
           out_w, out_b):
    del batch
    h = jnp.maximum(x @ emb_w + emb_b, 0.0)
    params = [(ec0_w1, ec0_b1, ec0_w2, ec0_b2),
              (ec1_w1, ec1_b1, ec1_w2, ec1_b2),
              (ec2_w1, ec2_b1, ec2_w2, ec2_b2),
              (ec3_w1, ec3_b1, ec3_w2, ec3_b2)]
    outs = []
    for (w1, b1, w2, b2) in params:
        h_pad = jnp.pad(h, ((0, PADN - N), (0, 0)))
        sq = jnp.sum(h_pad * h_pad, axis=1)
        sq_pad = jnp.where(jnp.arange(PADN) < N, sq, BIG)[None, :]
        idx = _knn_idx_pallas(h_pad, sq_pad)
        w1a, w1b = w1[:H], w1[H:]
        a = h_pad @ (w1a - w1b) + b1
        g = h_pad @ w1b
        idx3 = idx.reshape(NW, NCH, CH)
        s = _sc_edge(idx3, a, g)
        h = _h_update_pallas(s, w2, b2)[:N]
        outs.append(h)
    hc = jnp.concatenate(outs, axis=-1)
    return pl.pallas_call(
        _final_body,
        out_shape=jax.ShapeDtypeStruct((N, 3), jnp.float32),
    )(hc, out_w, out_b[None, :])


# speedup vs baseline: 1.5820x; 1.5820x over previous
import functools

import jax
import jax.numpy as jnp
from jax import lax
from jax.experimental import pallas as pl
from jax.experimental.pallas import tpu as pltpu
from jax.experimental.pallas import tpu_sc as plsc

N = 10000
H = 16
K = 16
L = 4
RB = 256
PADN = 10240
BIG = 1.0e30


NLANE = 128
NG = PADN // NLANE
T = 5


def _knn_body(nrows, hr_ref, hT_ref, sq_ref, idx_ref, d_ref):
    i = pl.program_id(0)
    hr = hr_ref[...]
    p = jnp.dot(hr, hT_ref[...], preferred_element_type=jnp.float32)
    sq_r = jnp.sum(hr * hr, axis=1)
    d = (sq_r[:, None] + sq_ref[...]) - 2.0 * p
    col = jax.lax.broadcasted_iota(jnp.int32, (RB, PADN), 1)
    row = jax.lax.broadcasted_iota(jnp.int32, (RB, PADN), 0) + i * RB
    d = jnp.where(col == row, BIG, d)
    d_ref[...] = d

    RG = 32
    lane32 = jax.lax.broadcasted_iota(jnp.int32, (RG, NLANE), 1)
    S_parts, C_parts = [], []
    for rg in range(RB // RG):
        init = ([jnp.full((RG, NLANE), BIG, jnp.float32) for _ in range(T)]
                + [jnp.zeros((RG, NLANE), jnp.int32) for _ in range(T)])

        def chunk_step(ci, st):
            ms, gs = list(st[:T]), list(st[T:])
            v = d_ref[pl.ds(rg * RG, RG), pl.ds(ci * NLANE, NLANE)]
            vc = ci * NLANE + lane32
            for t in range(T):
                sw = v < ms[t]
                nv = jnp.minimum(ms[t], v)
                cr = jnp.maximum(ms[t], v)
                ng = jnp.where(sw, vc, gs[t])
                crg = jnp.where(sw, gs[t], vc)
                ms[t], gs[t] = nv, ng
                v, vc = cr, crg
            return tuple(ms) + tuple(gs)

        st = jax.lax.fori_loop(0, NG, chunk_step, tuple(init))
        S_parts.append(jnp.stack(st[:T], axis=1))
        C_parts.append(jnp.stack(st[T:], axis=1))
    W2 = T * NLANE
    S = jnp.concatenate(S_parts, axis=0).reshape(RB, W2)
    C = jnp.concatenate(C_parts, axis=0).reshape(RB, W2)

    slot = jax.lax.broadcasted_iota(jnp.int32, (RB, W2), 1)
    idxs = []
    maxlev = jnp.zeros((RB,), jnp.int32)
    for _ in range(K):
        m = jnp.min(S, axis=1)
        first = jnp.min(jnp.where(S == m[:, None], slot, W2), axis=1)
        cp = jnp.min(jnp.where(slot == first[:, None], C, PADN), axis=1)
        idxs.append(cp)
        maxlev = jnp.maximum(maxlev, first // NLANE)
        S = jnp.where(slot == first[:, None], BIG, S)
    danger = jnp.max(maxlev) >= T - 1

    @pl.when(jnp.logical_not(danger))
    def _fast():
        idx_ref[...] = jnp.stack(idxs, axis=1).astype(jnp.int32)

    @pl.when(danger)
    def _slow():
        res = []
        for _ in range(K):
            dd = d_ref[...]
            mm = jnp.min(dd, axis=1)
            ff = jnp.min(jnp.where(dd == mm[:, None], col, PADN), axis=1)
            res.append(ff)
            d_ref[...] = jnp.where(col == ff[:, None], BIG, dd)
        idx_ref[...] = jnp.stack(res, axis=1).astype(jnp.int32)


def _knn_idx_pallas(h_pad, sq_pad):
    grid = PADN // RB
    body = functools.partial(_knn_body, N)
    return pl.pallas_call(
        body,
        grid=(grid,),
        in_specs=[
            pl.BlockSpec((RB, H), lambda i: (i, 0)),
            pl.BlockSpec((H, PADN), lambda i: (0, 0)),
            pl.BlockSpec((1, PADN), lambda i: (0, 0)),
        ],
        out_specs=pl.BlockSpec((RB, K), lambda i: (i, 0)),
        out_shape=jax.ShapeDtypeStruct((PADN, K), jnp.int32),
        scratch_shapes=[pltpu.VMEM((RB, PADN), jnp.float32)],
    )(h_pad, h_pad.T, sq_pad)


NW = 32
ROWS_W = PADN // NW
CH = 128
NCH = ROWS_W * K // CH


def _sc_edge(idx3, a, g):
    mesh = plsc.VectorSubcoreMesh(core_axis_name="c", subcore_axis_name="s")

    @functools.partial(
        pl.kernel, mesh=mesh,
        out_type=jax.ShapeDtypeStruct((PADN, H), jnp.float32),
        compiler_params=pltpu.CompilerParams(use_tc_tiling_on_sc=False),
        scratch_types=[
            pltpu.VMEM((NCH, CH), jnp.int32),
            pltpu.VMEM((ROWS_W * K, H), jnp.float32),
            pltpu.VMEM((ROWS_W, H), jnp.float32),
            pltpu.VMEM((ROWS_W, H), jnp.float32),
            pltpu.SemaphoreType.DMA,
        ],
    )
    def k(idx_hbm, a_hbm, g_hbm, out_hbm, idx_v, rows_v, a_v, s_v, sem):
        wid = lax.axis_index("s") * 2 + lax.axis_index("c")
        base = wid * ROWS_W
        pltpu.sync_copy(idx_hbm.at[wid], idx_v)
        pltpu.sync_copy(a_hbm.at[pl.ds(base, ROWS_W)], a_v)

        def gat(c, carry):
            pltpu.async_copy(g_hbm.at[idx_v.at[c]],
                             rows_v.at[pl.ds(c * CH, CH)], sem).wait()
            return carry

        lax.fori_loop(0, NCH, gat, 0)

        def row(r, carry):
            a_r = a_v[r, :]
            s = jnp.maximum(a_r + rows_v[r * K, :], 0.0)
            for j in range(1, K):
                s = s + jnp.maximum(a_r + rows_v[r * K + j, :], 0.0)
            s_v[r, :] = s
            return carry

        lax.fori_loop(0, ROWS_W, row, 0)
        pltpu.sync_copy(s_v, out_hbm.at[pl.ds(base, ROWS_W)])

    return k(idx3, a, g)


def _hupd_body(s_ref, w2_ref, b2_ref, o_ref):
    o_ref[...] = jnp.dot(s_ref[...] * (1.0 / K), w2_ref[...],
                         preferred_element_type=jnp.float32) + b2_ref[...]


def _h_update_pallas(s, w2, b2):
    return pl.pallas_call(
        _hupd_body,
        out_shape=jax.ShapeDtypeStruct((PADN, H), jnp.float32),
    )(s, w2, b2[None, :])


def _final_body(hc_ref, w_ref, b_ref, o_ref):
    o_ref[...] = jnp.dot(hc_ref[...], w_ref[...],
                         preferred_element_type=jnp.float32) + b_ref[...]


def kernel(x, batch, emb_w, emb_b,
           ec0_w1, ec0_b1, ec0_w2, ec0_b2,
           ec1_w1, ec1_b1, ec1_w2, ec1_b2,
           ec2_w1, ec2_b1, ec2_w2, ec2_b2,
           ec3_w1, ec3_b1, ec3_w2, ec3_b2,
           out_w, out_b):
    del batch
    h = jnp.maximum(x @ emb_w + emb_b, 0.0)
    params = [(ec0_w1, ec0_b1, ec0_w2, ec0_b2),
              (ec1_w1, ec1_b1, ec1_w2, ec1_b2),
              (ec2_w1, ec2_b1, ec2_w2, ec2_b2),
              (ec3_w1, ec3_b1, ec3_w2, ec3_b2)]
    outs = []
    for (w1, b1, w2, b2) in params:
        h_pad = jnp.pad(h, ((0, PADN - N), (0, 0)))
        sq = jnp.sum(h_pad * h_pad, axis=1)
        sq_pad = jnp.where(jnp.arange(PADN) < N, sq, BIG)[None, :]
        idx = _knn_idx_pallas(h_pad, sq_pad)
        w1a, w1b = w1[:H], w1[H:]
        a = h_pad @ (w1a - w1b) + b1
        g = h_pad @ w1b
        idx3 = idx.reshape(NW, NCH, CH)
        s = _sc_edge(idx3, a, g)
        h = _h_update_pallas(s, w2, b2)[:N]
        outs.append(h)
    hc = jnp.concatenate(outs, axis=-1)
    return pl.pallas_call(
        _final_body,
        out_shape=jax.ShapeDtypeStruct((N, 3), jnp.float32),
    )(hc, out_w, out_b[None, :])


# speedup vs baseline: 1.6395x; 1.0363x over previous
import functools

import jax
import jax.numpy as jnp
from jax import lax
from jax.experimental import pallas as pl
from jax.experimental.pallas import tpu as pltpu
from jax.experimental.pallas import tpu_sc as plsc

N = 10000
H = 16
K = 16
L = 4
RB = 256
PADN = 10240
BIG = 1.0e30


NLANE = 128
NG = PADN // NLANE
T = 5


def _knn_body(nrows, hr_ref, hT_ref, idx_ref, d_ref):
    i = pl.program_id(0)
    hr = hr_ref[...]
    sq_r = jnp.sum(hr * hr, axis=1)
    hr_aug = jnp.concatenate(
        [hr, jnp.ones((RB, 1), jnp.float32), sq_r[:, None]], axis=1)
    d_ref[...] = jnp.dot(hr_aug, hT_ref[...],
                         preferred_element_type=jnp.float32)

    RG = 32
    lane32 = jax.lax.broadcasted_iota(jnp.int32, (RG, NLANE), 1)
    S_parts, C_parts = [], []
    for rg in range(RB // RG):
        init = ([jnp.full((RG, NLANE), BIG, jnp.float32) for _ in range(T)]
                + [jnp.zeros((RG, NLANE), jnp.int32) for _ in range(T)])

        def chunk_step(ci, st):
            ms, gs = list(st[:T]), list(st[T:])
            v = d_ref[pl.ds(rg * RG, RG), pl.ds(ci * NLANE, NLANE)]
            vc = ci * NLANE + lane32
            for t in range(T):
                sw = v < ms[t]
                nv = jnp.minimum(ms[t], v)
                cr = jnp.maximum(ms[t], v)
                ng = jnp.where(sw, vc, gs[t])
                crg = jnp.where(sw, gs[t], vc)
                ms[t], gs[t] = nv, ng
                v, vc = cr, crg
            return tuple(ms) + tuple(gs)

        st = jax.lax.fori_loop(0, NG, chunk_step, tuple(init))
        S_parts.append(jnp.stack(st[:T], axis=1))
        C_parts.append(jnp.stack(st[T:], axis=1))
    W2 = T * NLANE
    S = jnp.concatenate(S_parts, axis=0).reshape(RB, W2)
    C = jnp.concatenate(C_parts, axis=0).reshape(RB, W2)

    rowid = i * RB + jax.lax.broadcasted_iota(jnp.int32, (RB, W2), 0)
    S = jnp.where(C == rowid, BIG, S)
    slot = jax.lax.broadcasted_iota(jnp.int32, (RB, W2), 1)
    idxs = []
    maxlev = jnp.zeros((RB,), jnp.int32)
    for _ in range(K):
        m = jnp.min(S, axis=1)
        first = jnp.min(jnp.where(S == m[:, None], slot, W2), axis=1)
        cp = jnp.min(jnp.where(slot == first[:, None], C, PADN), axis=1)
        idxs.append(cp)
        maxlev = jnp.maximum(maxlev, first // NLANE)
        S = jnp.where(slot == first[:, None], BIG, S)
    danger = jnp.max(maxlev) >= T - 1

    @pl.when(jnp.logical_not(danger))
    def _fast():
        idx_ref[...] = jnp.stack(idxs, axis=1).astype(jnp.int32)

    @pl.when(danger)
    def _slow():
        col = jax.lax.broadcasted_iota(jnp.int32, (RB, PADN), 1)
        row = jax.lax.broadcasted_iota(jnp.int32, (RB, PADN), 0) + i * RB
        d_ref[...] = jnp.where(col == row, BIG, d_ref[...])
        res = []
        for _ in range(K):
            dd = d_ref[...]
            mm = jnp.min(dd, axis=1)
            ff = jnp.min(jnp.where(dd == mm[:, None], col, PADN), axis=1)
            res.append(ff)
            d_ref[...] = jnp.where(col == ff[:, None], BIG, dd)
        idx_ref[...] = jnp.stack(res, axis=1).astype(jnp.int32)


HAUG = H + 2


def _knn_idx_pallas(h_pad, sq_pad):
    grid = PADN // RB
    body = functools.partial(_knn_body, N)
    hT_aug = jnp.concatenate(
        [-2.0 * h_pad.T, sq_pad, jnp.ones((1, PADN), jnp.float32)], axis=0)
    return pl.pallas_call(
        body,
        grid=(grid,),
        in_specs=[
            pl.BlockSpec((RB, H), lambda i: (i, 0)),
            pl.BlockSpec((HAUG, PADN), lambda i: (0, 0)),
        ],
        out_specs=pl.BlockSpec((RB, K), lambda i: (i, 0)),
        out_shape=jax.ShapeDtypeStruct((PADN, K), jnp.int32),
        scratch_shapes=[pltpu.VMEM((RB, PADN), jnp.float32)],
    )(h_pad, hT_aug)


NW = 32
ROWS_W = PADN // NW
CH = 128
NCH = ROWS_W * K // CH


def _sc_edge(idx3, a, g):
    mesh = plsc.VectorSubcoreMesh(core_axis_name="c", subcore_axis_name="s")

    @functools.partial(
        pl.kernel, mesh=mesh,
        out_type=jax.ShapeDtypeStruct((PADN, H), jnp.float32),
        compiler_params=pltpu.CompilerParams(use_tc_tiling_on_sc=False),
        scratch_types=[
            pltpu.VMEM((NCH, CH), jnp.int32),
            pltpu.VMEM((ROWS_W * K, H), jnp.float32),
            pltpu.VMEM((ROWS_W, H), jnp.float32),
            pltpu.VMEM((ROWS_W, H), jnp.float32),
            pltpu.SemaphoreType.DMA,
        ],
    )
    def k(idx_hbm, a_hbm, g_hbm, out_hbm, idx_v, rows_v, a_v, s_v, sem):
        wid = lax.axis_index("s") * 2 + lax.axis_index("c")
        base = wid * ROWS_W
        pltpu.sync_copy(idx_hbm.at[wid], idx_v)
        pltpu.sync_copy(a_hbm.at[pl.ds(base, ROWS_W)], a_v)

        def gat(c, carry):
            pltpu.async_copy(g_hbm.at[idx_v.at[c]],
                             rows_v.at[pl.ds(c * CH, CH)], sem).wait()
            return carry

        lax.fori_loop(0, NCH, gat, 0)

        def row(r, carry):
            a_r = a_v[r, :]
            s = jnp.maximum(a_r + rows_v[r * K, :], 0.0)
            for j in range(1, K):
                s = s + jnp.maximum(a_r + rows_v[r * K + j, :], 0.0)
            s_v[r, :] = s
            return carry

        lax.fori_loop(0, ROWS_W, row, 0)
        pltpu.sync_copy(s_v, out_hbm.at[pl.ds(base, ROWS_W)])

    return k(idx3, a, g)


def _hupd_body(s_ref, w2_ref, b2_ref, o_ref):
    o_ref[...] = jnp.dot(s_ref[...] * (1.0 / K), w2_ref[...],
                         preferred_element_type=jnp.float32) + b2_ref[...]


def _h_update_pallas(s, w2, b2):
    return pl.pallas_call(
        _hupd_body,
        out_shape=jax.ShapeDtypeStruct((PADN, H), jnp.float32),
    )(s, w2, b2[None, :])


def _final_body(hc_ref, w_ref, b_ref, o_ref):
    o_ref[...] = jnp.dot(hc_ref[...], w_ref[...],
                         preferred_element_type=jnp.float32) + b_ref[...]


def kernel(x, batch, emb_w, emb_b,
           ec0_w1, ec0_b1, ec0_w2, ec0_b2,
           ec1_w1, ec1_b1, ec1_w2, ec1_b2,
           ec2_w1, ec2_b1, ec2_w2, ec2_b2,
           ec3_w1, ec3_b1, ec3_w2, ec3_b2,
           out_w, out_b):
    del batch
    h = jnp.maximum(x @ emb_w + emb_b, 0.0)
    params = [(ec0_w1, ec0_b1, ec0_w2, ec0_b2),
              (ec1_w1, ec1_b1, ec1_w2, ec1_b2),
              (ec2_w1, ec2_b1, ec2_w2, ec2_b2),
              (ec3_w1, ec3_b1, ec3_w2, ec3_b2)]
    outs = []
    for (w1, b1, w2, b2) in params:
        h_pad = jnp.pad(h, ((0, PADN - N), (0, 0)))
        sq = jnp.sum(h_pad * h_pad, axis=1)
        sq_pad = jnp.where(jnp.arange(PADN) < N, sq, BIG)[None, :]
        idx = _knn_idx_pallas(h_pad, sq_pad)
        w1a, w1b = w1[:H], w1[H:]
        a = h_pad @ (w1a - w1b) + b1
        g = h_pad @ w1b
        idx3 = idx.reshape(NW, NCH, CH)
        s = _sc_edge(idx3, a, g)
        h = _h_update_pallas(s, w2, b2)[:N]
        outs.append(h)
    hc = jnp.concatenate(outs, axis=-1)
    return pl.pallas_call(
        _final_body,
        out_shape=jax.ShapeDtypeStruct((N, 3), jnp.float32),
    )(hc, out_w, out_b[None, :])
